# Initial kernel scaffold; baseline (speedup 1.0000x reference)
import jax
import jax.numpy as jnp
from jax.experimental import pallas as pl


def kernel(x, edge_index, W1, b1, W2, b2):
    raise NotImplementedError("write your pallas kernel here")


# speedup vs baseline: 44.7343x; 44.7343x over previous
import functools

import jax
import jax.numpy as jnp
from jax import lax
from jax.experimental import pallas as pl
from jax.experimental.pallas import tpu as pltpu
from jax.experimental.pallas import tpu_sc as plsc

N = 10000
NPAD = 10240
E = 320000
IN_DIM = 128
HID = 16
NCLS = 40

NC = 2
NS = 16
NW = NC * NS
W_E = 10368
EP = W_E * NW
K_W = W_E // 128
CH = 9
NCH = K_W // CH
ROWS_T = NPAD // NS

RB = 1024
GRID = NPAD // RB

_f32 = jnp.float32


def _make_sc_deg():
  mesh = plsc.VectorSubcoreMesh(core_axis_name="c", subcore_axis_name="s")
  scratch = [
      pltpu.VMEM((K_W, 128), jnp.int32),
      pltpu.VMEM((128,), _f32),
      pltpu.VMEM_SHARED((NPAD,), _f32),
  ]

  @functools.partial(
      pl.kernel,
      out_type=jax.ShapeDtypeStruct((NC, NPAD), _f32),
      mesh=mesh,
      scratch_types=scratch,
  )
  def sc_deg(rowi, z1, ones, degp, ridx, ones_v, acc):
    ci = lax.axis_index("c")
    si = lax.axis_index("s")
    wid = si * NC + ci
    t0 = si * ROWS_T
    pltpu.sync_copy(z1.at[pl.ds(t0, ROWS_T)], acc.at[pl.ds(t0, ROWS_T)])
    pltpu.sync_copy(ones, ones_v)
    pltpu.sync_copy(rowi.at[wid], ridx)
    plsc.subcore_barrier()

    def step(j, carry):
      pltpu.sync_copy(ones_v, acc.at[ridx.at[j]], add=True)
      return carry

    lax.fori_loop(0, K_W, step, 0)
    plsc.subcore_barrier()
    pltpu.sync_copy(acc.at[pl.ds(t0, ROWS_T)], degp.at[ci, pl.ds(t0, ROWS_T)])

  return sc_deg


def _make_sc_agg(with_t: bool):
  mesh = plsc.VectorSubcoreMesh(core_axis_name="c", subcore_axis_name="s")
  out_type = (
      jax.ShapeDtypeStruct((NC, NPAD, HID), _f32),
      jax.ShapeDtypeStruct((NC, NPAD), _f32),
  )
  scratch = [
      pltpu.VMEM((K_W, 128), jnp.int32),
      pltpu.VMEM((K_W, 128), jnp.int32),
      pltpu.VMEM((CH * 128, HID), _f32),
      pltpu.VMEM((NPAD,), _f32),
      pltpu.VMEM((CH * 128,), _f32),
      pltpu.VMEM_SHARED((NPAD, HID), _f32),
      pltpu.VMEM_SHARED((NPAD,), _f32),
      pltpu.VMEM_SHARED((NPAD, HID), _f32),
      pltpu.SemaphoreType.DMA,
  ]

  @functools.partial(pl.kernel, out_type=out_type, mesh=mesh,
                     scratch_types=scratch,
                     compiler_params=pltpu.CompilerParams(
                         needs_layout_passes=False,
                         use_tc_tiling_on_sc=False))
  def sc_agg(tbl, rowi, coli, dis, z2, z1, aggp, tp,
             ridx, cidx, rows_v, dis_v, tbuf, acc, tacc, tstage, sem):
    ci = lax.axis_index("c")
    si = lax.axis_index("s")
    wid = si * NC + ci
    t0 = si * ROWS_T
    pltpu.sync_copy(z2.at[pl.ds(t0, ROWS_T)], acc.at[pl.ds(t0, ROWS_T)])
    pltpu.sync_copy(z1.at[pl.ds(t0, ROWS_T)], tacc.at[pl.ds(t0, ROWS_T)])
    pltpu.sync_copy(tbl.at[pl.ds(t0, ROWS_T)], tstage.at[pl.ds(t0, ROWS_T)])
    pltpu.sync_copy(rowi.at[wid], ridx)
    pltpu.sync_copy(coli.at[wid], cidx)
    if with_t:
      pltpu.sync_copy(dis, dis_v)
    plsc.subcore_barrier()

    def chunk(cb, carry):
      base = cb * CH
      cps = [
          pltpu.async_copy(tstage.at[ridx.at[base + j]],
                           rows_v.at[pl.ds(j * 128, 128)], sem)
          for j in range(CH)
      ]
      for cp in cps:
        cp.wait()
      if with_t:
        def tstep(m, tc):
          j = m // 8
          off = (m % 8) * 16
          idx16 = ridx[base + j, pl.ds(off, 16)]
          tbuf[pl.ds(m * 16, 16)] = plsc.load_gather(dis_v, [idx16])
          return tc
        lax.fori_loop(0, CH * 8, tstep, 0)
      for j in range(CH):
        pltpu.sync_copy(rows_v.at[pl.ds(j * 128, 128)],
                        acc.at[cidx.at[base + j]], add=True)
      if with_t:
        for j in range(CH):
          pltpu.sync_copy(tbuf.at[pl.ds(j * 128, 128)],
                          tacc.at[cidx.at[base + j]], add=True)
      return carry

    lax.fori_loop(0, NCH, chunk, 0)
    plsc.subcore_barrier()
    pltpu.sync_copy(acc.at[pl.ds(t0, ROWS_T)], aggp.at[ci, pl.ds(t0, ROWS_T)])
    pltpu.sync_copy(tacc.at[pl.ds(t0, ROWS_T)], tp.at[ci, pl.ds(t0, ROWS_T)])

  return sc_agg


_sc_deg = _make_sc_deg()
_sc_agg_t = _make_sc_agg(True)
_sc_agg = _make_sc_agg(False)


def _tc1_body(x_blk, w1t, b1, degp, h1s_out, dis_out):
  i = pl.program_id(0)
  h = jnp.dot(x_blk[...], w1t[...], preferred_element_type=_f32) + b1[...]
  deg = degp[0] + degp[1]
  rid = lax.broadcasted_iota(jnp.int32, (RB, 1), 0) + i * RB
  dis = jnp.where(rid < N, lax.rsqrt(deg), 0.0)
  h1s_out[...] = dis * h
  dis_out[...] = dis


def _tc1(xp, w1t, b1, degp):
  return pl.pallas_call(
      _tc1_body,
      grid=(GRID,),
      in_specs=[
          pl.BlockSpec((RB, IN_DIM), lambda i: (i, 0)),
          pl.BlockSpec((IN_DIM, HID), lambda i: (0, 0)),
          pl.BlockSpec((1, HID), lambda i: (0, 0)),
          pl.BlockSpec((NC, RB, 1), lambda i: (0, i, 0)),
      ],
      out_specs=[
          pl.BlockSpec((RB, HID), lambda i: (i, 0)),
          pl.BlockSpec((RB, 1), lambda i: (i, 0)),
      ],
      out_shape=(
          jax.ShapeDtypeStruct((NPAD, HID), _f32),
          jax.ShapeDtypeStruct((NPAD, 1), _f32),
      ),
  )(xp, w1t, b1, degp)


def _tc2_body(aggp, tp, dis, g_out, s_out):
  a = aggp[0] + aggp[1]
  t = tp[0] + tp[1]
  d = dis[...]
  out1 = d * a
  g_out[...] = d * jnp.maximum(out1, 0.0)
  s_out[...] = d * t


def _tc2(aggp, tp, dis):
  return pl.pallas_call(
      _tc2_body,
      grid=(GRID,),
      in_specs=[
          pl.BlockSpec((NC, RB, HID), lambda i: (0, i, 0)),
          pl.BlockSpec((NC, RB, 1), lambda i: (0, i, 0)),
          pl.BlockSpec((RB, 1), lambda i: (i, 0)),
      ],
      out_specs=[
          pl.BlockSpec((RB, HID), lambda i: (i, 0)),
          pl.BlockSpec((RB, 1), lambda i: (i, 0)),
      ],
      out_shape=(
          jax.ShapeDtypeStruct((NPAD, HID), _f32),
          jax.ShapeDtypeStruct((NPAD, 1), _f32),
      ),
  )(aggp, tp, dis)


def _tc3_body(aggp, dis, s, w2t, b2, out):
  a = (aggp[0] + aggp[1]) * dis[...]
  o = jnp.dot(a, w2t[...], preferred_element_type=_f32) + s[...] * b2[...]
  m = jnp.max(o, axis=1, keepdims=True)
  lse = jnp.log(jnp.sum(jnp.exp(o - m), axis=1, keepdims=True)) + m
  out[...] = o - lse


def _tc3(aggp, dis, s, w2t, b2):
  return pl.pallas_call(
      _tc3_body,
      grid=(GRID,),
      in_specs=[
          pl.BlockSpec((NC, RB, HID), lambda i: (0, i, 0)),
          pl.BlockSpec((RB, 1), lambda i: (i, 0)),
          pl.BlockSpec((RB, 1), lambda i: (i, 0)),
          pl.BlockSpec((HID, NCLS), lambda i: (0, 0)),
          pl.BlockSpec((1, NCLS), lambda i: (0, 0)),
      ],
      out_specs=pl.BlockSpec((RB, NCLS), lambda i: (i, 0)),
      out_shape=jax.ShapeDtypeStruct((N, NCLS), _f32),
  )(aggp, dis, s, w2t, b2)


def kernel(x, edge_index, W1, b1, W2, b2):
  ei = edge_index.astype(jnp.int32)
  loops = jnp.arange(N, dtype=jnp.int32)
  npadfill = EP - (E + N)
  padi = N + (jnp.arange(npadfill, dtype=jnp.int32) % (NPAD - N))
  row = jnp.concatenate([ei[0], loops, padi]).reshape(NW, K_W, 128)
  col = jnp.concatenate([ei[1], loops, padi]).reshape(NW, K_W, 128)

  z1 = jnp.zeros((NPAD,), _f32)
  z2 = jnp.zeros((NPAD, HID), _f32)
  ones = jnp.ones((128,), _f32)
  xp = jnp.pad(x, ((0, NPAD - N), (0, 0)))

  degp = _sc_deg(row, z1, ones)
  h1s, dis2 = _tc1(xp, W1.T, b1.reshape(1, HID), degp.reshape(NC, NPAD, 1))
  dis1 = dis2.reshape(NPAD)
  agg1, t = _sc_agg_t(h1s, row, col, dis1, z2, z1)
  g, s = _tc2(agg1, t.reshape(NC, NPAD, 1), dis2)
  agg2, _ = _sc_agg(g, row, col, dis1, z2, z1)
  return _tc3(agg2, dis2, s, W2.T, b2.reshape(1, NCLS))


# speedup vs baseline: 55.4930x; 1.2405x over previous
import functools

import jax
import jax.numpy as jnp
from jax import lax
from jax.experimental import pallas as pl
from jax.experimental.pallas import tpu as pltpu
from jax.experimental.pallas import tpu_sc as plsc

N = 10000
NPAD = 10240
E = 320000
IN_DIM = 128
HID = 16
NCLS = 40

NC = 2
NS = 16
NW = NC * NS
W_E = 10368
EP = W_E * NW
K_W = W_E // 128
CH = 9
NCH = K_W // CH
ROWS_T = NPAD // NS

RB = 1024
GRID = NPAD // RB

_f32 = jnp.float32

_SC_PARAMS = pltpu.CompilerParams(needs_layout_passes=False,
                                  use_tc_tiling_on_sc=False)


def _frsqrt(d):
  xi = plsc.bitcast(d, jnp.int32)
  y = plsc.bitcast(jnp.int32(0x5F3759DF) - lax.shift_right_logical(xi, 1),
                   _f32)
  for _ in range(3):
    y = y * (1.5 - 0.5 * d * y * y)
  return y


def _make_sc_b():
  mesh = plsc.VectorSubcoreMesh(core_axis_name="c", subcore_axis_name="s")
  out_type = (
      jax.ShapeDtypeStruct((NC, NPAD, HID), _f32),
      jax.ShapeDtypeStruct((NC, NPAD), _f32),
      jax.ShapeDtypeStruct((NPAD,), _f32),
  )
  scratch = [
      pltpu.VMEM((K_W, 128), jnp.int32),
      pltpu.VMEM((K_W, 128), jnp.int32),
      pltpu.VMEM((K_W, 128), jnp.int32),
      pltpu.VMEM((CH * 128, HID), _f32),
      pltpu.VMEM((CH * 128, HID), _f32),
      pltpu.VMEM((NPAD,), _f32),
      pltpu.VMEM((CH * 128,), _f32),
      pltpu.VMEM((CH * 128,), _f32),
      pltpu.VMEM((ROWS_T, HID), _f32),
      pltpu.VMEM((ROWS_T,), _f32),
      pltpu.VMEM((128,), _f32),
      pltpu.VMEM_SHARED((NPAD, HID), _f32),
      pltpu.VMEM_SHARED((NPAD,), _f32),
      pltpu.VMEM_SHARED((NPAD,), _f32),
      pltpu.VMEM_SHARED((NPAD,), _f32),
      pltpu.VMEM_SHARED((NPAD, HID), _f32),
      pltpu.SemaphoreType.DMA,
      pltpu.SemaphoreType.DMA,
      pltpu.SemaphoreType.DMA,
      pltpu.SemaphoreType.DMA,
      pltpu.SemaphoreType.DMA,
  ]

  @functools.partial(pl.kernel, out_type=out_type, mesh=mesh,
                     scratch_types=scratch, compiler_params=_SC_PARAMS)
  def sc_b(h1, rowi, coli, z1, z2, ones, aggp, tp, dis_out,
           ridx, cidx, didx, rowsA, rowsB, dis_v, tbufA, tbufB, hloc, dloc,
           ones_v, acc, tacc, dacc, dis_sh, tstage,
           gsemA, gsemB, ssemA, ssemB, dsem):
    ci = lax.axis_index("c")
    si = lax.axis_index("s")
    wid = si * NC + ci
    t0 = si * ROWS_T
    sl = pl.ds(t0, ROWS_T)
    pltpu.sync_copy(z2.at[sl], acc.at[sl])
    pltpu.sync_copy(z1.at[sl], tacc.at[sl])
    pltpu.sync_copy(z1.at[sl], dacc.at[sl])
    pltpu.sync_copy(ones, ones_v)
    pltpu.sync_copy(rowi.at[wid], ridx)
    pltpu.sync_copy(coli.at[wid], cidx)
    plsc.subcore_barrier()

    for w_off in range(2):
      pltpu.sync_copy(rowi.at[2 * si + w_off], didx)
      for g in range(3):
        dps = [
            pltpu.async_copy(ones_v, dacc.at[didx.at[g * 27 + j]], dsem,
                             add=True)
            for j in range(27)
        ]
        for d in dps:
          d.wait()
    plsc.subcore_barrier()

    pltpu.sync_copy(dacc.at[sl], dloc)

    def dis_step(m, carry):
      d = dloc[pl.ds(m * 16, 16)]
      rid = lax.broadcasted_iota(jnp.int32, (16,), 0) + (t0 + m * 16)
      dloc[pl.ds(m * 16, 16)] = jnp.where(rid < N, _frsqrt(d), 0.0)
      return carry

    lax.fori_loop(0, ROWS_T // 16, dis_step, 0)
    pltpu.sync_copy(dloc, dis_sh.at[sl])

    pltpu.sync_copy(h1.at[sl], hloc)

    def scale_step(r, carry):
      splat = plsc.load_gather(
          dloc, [jnp.zeros((16,), jnp.int32) + r])
      hloc[r, :] = hloc[r, :] * splat
      return carry

    lax.fori_loop(0, ROWS_T, scale_step, 0)
    pltpu.sync_copy(hloc, tstage.at[sl])
    plsc.subcore_barrier()
    pltpu.sync_copy(dis_sh, dis_v)

    def issue_gathers(c, rb, gs):
      base = c * CH
      return [
          pltpu.async_copy(tstage.at[ridx.at[base + j]],
                           rb.at[pl.ds(j * 128, 128)], gs)
          for j in range(CH)
      ]

    def tloop(c, tb):
      base = c * CH

      def tstep(m, carry):
        j = m // 8
        off = (m % 8) * 16
        idx16 = ridx[base + j, pl.ds(off, 16)]
        tb[pl.ds(m * 16, 16)] = plsc.load_gather(dis_v, [idx16])
        return carry

      lax.fori_loop(0, CH * 8, tstep, 0)

    def issue_scatters(c, rb, tb, ss):
      base = c * CH
      out = []
      for j in range(CH):
        out.append(pltpu.async_copy(rb.at[pl.ds(j * 128, 128)],
                                    acc.at[cidx.at[base + j]], ss, add=True))
        out.append(pltpu.async_copy(tb.at[pl.ds(j * 128, 128)],
                                    tacc.at[cidx.at[base + j]], ss, add=True))
      return out

    bufs = [(rowsA, tbufA, gsemA, ssemA), (rowsB, tbufB, gsemB, ssemB)]
    pend_g = {0: issue_gathers(0, bufs[0][0], bufs[0][2])}
    pend_s = {}
    for c in range(NCH):
      rb, tb, gs, ss = bufs[c % 2]
      if c + 1 < NCH:
        nrb, _, ngs, _ = bufs[(c + 1) % 2]
        if c - 1 in pend_s:
          for d in pend_s.pop(c - 1):
            d.wait()
        pend_g[c + 1] = issue_gathers(c + 1, nrb, ngs)
      for d in pend_g.pop(c):
        d.wait()
      tloop(c, tb)
      pend_s[c] = issue_scatters(c, rb, tb, ss)
    for c in sorted(pend_s):
      for d in pend_s[c]:
        d.wait()

    plsc.subcore_barrier()
    pltpu.sync_copy(acc.at[sl], aggp.at[ci, sl])
    pltpu.sync_copy(tacc.at[sl], tp.at[ci, sl])

    @pl.when(ci == 0)
    def _():
      pltpu.sync_copy(dloc, dis_out.at[sl])

  return sc_b


def _make_sc_c():
  mesh = plsc.VectorSubcoreMesh(core_axis_name="c", subcore_axis_name="s")
  out_type = jax.ShapeDtypeStruct((NC, NPAD, HID), _f32)
  scratch = [
      pltpu.VMEM((K_W, 128), jnp.int32),
      pltpu.VMEM((K_W, 128), jnp.int32),
      pltpu.VMEM((CH * 128, HID), _f32),
      pltpu.VMEM((CH * 128, HID), _f32),
      pltpu.VMEM((ROWS_T, HID), _f32),
      pltpu.VMEM((ROWS_T, HID), _f32),
      pltpu.VMEM((ROWS_T,), _f32),
      pltpu.VMEM_SHARED((NPAD, HID), _f32),
      pltpu.VMEM_SHARED((NPAD, HID), _f32),
      pltpu.SemaphoreType.DMA,
      pltpu.SemaphoreType.DMA,
      pltpu.SemaphoreType.DMA,
      pltpu.SemaphoreType.DMA,
  ]

  @functools.partial(pl.kernel, out_type=out_type, mesh=mesh,
                     scratch_types=scratch, compiler_params=_SC_PARAMS)
  def sc_c(p, dis, rowi, coli, z2, aggp,
           ridx, cidx, rowsA, rowsB, g0, g1, dloc,
           acc, tstage, gsemA, gsemB, ssemA, ssemB):
    ci = lax.axis_index("c")
    si = lax.axis_index("s")
    wid = si * NC + ci
    t0 = si * ROWS_T
    sl = pl.ds(t0, ROWS_T)
    pltpu.sync_copy(z2.at[sl], acc.at[sl])
    pltpu.sync_copy(rowi.at[wid], ridx)
    pltpu.sync_copy(coli.at[wid], cidx)
    pltpu.sync_copy(p.at[0, sl], g0)
    pltpu.sync_copy(p.at[1, sl], g1)
    pltpu.sync_copy(dis.at[sl], dloc)

    def g_step(r, carry):
      splat = plsc.load_gather(dloc, [jnp.zeros((16,), jnp.int32) + r])
      v = (g0[r, :] + g1[r, :]) * splat
      g0[r, :] = jnp.maximum(v, 0.0) * splat
      return carry

    lax.fori_loop(0, ROWS_T, g_step, 0)
    pltpu.sync_copy(g0, tstage.at[sl])
    plsc.subcore_barrier()

    def issue_gathers(c, rb, gs):
      base = c * CH
      return [
          pltpu.async_copy(tstage.at[ridx.at[base + j]],
                           rb.at[pl.ds(j * 128, 128)], gs)
          for j in range(CH)
      ]

    def issue_scatters(c, rb, ss):
      base = c * CH
      return [
          pltpu.async_copy(rb.at[pl.ds(j * 128, 128)],
                           acc.at[cidx.at[base + j]], ss, add=True)
          for j in range(CH)
      ]

    bufs = [(rowsA, gsemA, ssemA), (rowsB, gsemB, ssemB)]
    pend_g = {0: issue_gathers(0, bufs[0][0], bufs[0][1])}
    pend_s = {}
    for c in range(NCH):
      rb, gs, ss = bufs[c % 2]
      if c + 1 < NCH:
        nrb, ngs, _ = bufs[(c + 1) % 2]
        if c - 1 in pend_s:
          for d in pend_s.pop(c - 1):
            d.wait()
        pend_g[c + 1] = issue_gathers(c + 1, nrb, ngs)
      for d in pend_g.pop(c):
        d.wait()
      pend_s[c] = issue_scatters(c, rb, ss)
    for c in sorted(pend_s):
      for d in pend_s[c]:
        d.wait()

    plsc.subcore_barrier()
    pltpu.sync_copy(acc.at[sl], aggp.at[ci, sl])

  return sc_c


_sc_b = _make_sc_b()
_sc_c = _make_sc_c()


def _tc_a_body(x_blk, w1t, b1, h1_out):
  h1_out[...] = jnp.dot(x_blk[...], w1t[...],
                        preferred_element_type=_f32) + b1[...]


def _tc_a(xp, w1t, b1):
  return pl.pallas_call(
      _tc_a_body,
      grid=(GRID,),
      in_specs=[
          pl.BlockSpec((RB, IN_DIM), lambda i: (i, 0)),
          pl.BlockSpec((IN_DIM, HID), lambda i: (0, 0)),
          pl.BlockSpec((1, HID), lambda i: (0, 0)),
      ],
      out_specs=pl.BlockSpec((RB, HID), lambda i: (i, 0)),
      out_shape=jax.ShapeDtypeStruct((NPAD, HID), _f32),
  )(xp, w1t, b1)


def _tc3_body(aggp, tp, dis, w2t, b2, out):
  d = dis[...]
  a = (aggp[0] + aggp[1]) * d
  s = (tp[0] + tp[1]) * d
  o = jnp.dot(a, w2t[...], preferred_element_type=_f32) + s * b2[...]
  m = jnp.max(o, axis=1, keepdims=True)
  lse = jnp.log(jnp.sum(jnp.exp(o - m), axis=1, keepdims=True)) + m
  out[...] = o - lse


def _tc3(aggp, tp, dis, w2t, b2):
  return pl.pallas_call(
      _tc3_body,
      grid=(GRID,),
      in_specs=[
          pl.BlockSpec((NC, RB, HID), lambda i: (0, i, 0)),
          pl.BlockSpec((NC, RB, 1), lambda i: (0, i, 0)),
          pl.BlockSpec((RB, 1), lambda i: (i, 0)),
          pl.BlockSpec((HID, NCLS), lambda i: (0, 0)),
          pl.BlockSpec((1, NCLS), lambda i: (0, 0)),
      ],
      out_specs=pl.BlockSpec((RB, NCLS), lambda i: (i, 0)),
      out_shape=jax.ShapeDtypeStruct((N, NCLS), _f32),
  )(aggp, tp, dis, w2t, b2)


def kernel(x, edge_index, W1, b1, W2, b2):
  ei = edge_index.astype(jnp.int32)
  loops = jnp.arange(N, dtype=jnp.int32)
  npadfill = EP - (E + N)
  padi = N + (jnp.arange(npadfill, dtype=jnp.int32) % (NPAD - N))
  row = jnp.concatenate([ei[0], loops, padi]).reshape(NW, K_W, 128)
  col = jnp.concatenate([ei[1], loops, padi]).reshape(NW, K_W, 128)

  z1 = jnp.zeros((NPAD,), _f32)
  z2 = jnp.zeros((NPAD, HID), _f32)
  ones = jnp.ones((128,), _f32)
  xp = jnp.pad(x, ((0, NPAD - N), (0, 0)))

  h1 = _tc_a(xp, W1.T, b1.reshape(1, HID))
  agg1, t, dis = _sc_b(h1, row, col, z1, z2, ones)
  agg2 = _sc_c(agg1, dis, row, col, z2)
  return _tc3(agg2, t.reshape(NC, NPAD, 1), dis.reshape(NPAD, 1),
              W2.T, b2.reshape(1, NCLS))


# speedup vs baseline: 56.3048x; 1.0146x over previous
import functools

import jax
import jax.numpy as jnp
from jax import lax
from jax.experimental import pallas as pl
from jax.experimental.pallas import tpu as pltpu
from jax.experimental.pallas import tpu_sc as plsc

N = 10000
NPAD = 10240
E = 320000
IN_DIM = 128
HID = 16
NCLS = 40

NC = 2
NS = 16
NW = NC * NS
W_E = 10368
EP = W_E * NW
K_W = W_E // 128
CH = 9
NCH = K_W // CH
ROWS_T = NPAD // NS

RB = 1024
GRID = NPAD // RB

_f32 = jnp.float32

_SC_PARAMS = pltpu.CompilerParams(needs_layout_passes=False,
                                  use_tc_tiling_on_sc=False)


def _frsqrt(d):
  xi = plsc.bitcast(d, jnp.int32)
  y = plsc.bitcast(jnp.int32(0x5F3759DF) - lax.shift_right_logical(xi, 1),
                   _f32)
  for _ in range(3):
    y = y * (1.5 - 0.5 * d * y * y)
  return y


def _make_sc_b():
  mesh = plsc.VectorSubcoreMesh(core_axis_name="c", subcore_axis_name="s")
  out_type = (
      jax.ShapeDtypeStruct((NC, NPAD, HID), _f32),
      jax.ShapeDtypeStruct((NC, NPAD), _f32),
      jax.ShapeDtypeStruct((NPAD,), _f32),
  )
  scratch = [
      pltpu.VMEM((W_E,), jnp.int32),
      pltpu.VMEM((K_W, 128), jnp.int32),
      pltpu.VMEM((K_W, 128), jnp.int32),
      pltpu.VMEM((CH * 128, HID), _f32),
      pltpu.VMEM((CH * 128, HID), _f32),
      pltpu.VMEM((CH * 128,), _f32),
      pltpu.VMEM((CH * 128,), _f32),
      pltpu.VMEM((ROWS_T, HID), _f32),
      pltpu.VMEM((ROWS_T,), _f32),
      pltpu.VMEM((K_W, 128), _f32),
      pltpu.VMEM_SHARED((NPAD, HID), _f32),
      pltpu.VMEM_SHARED((NPAD,), _f32),
      pltpu.VMEM_SHARED((NPAD,), _f32),
      pltpu.VMEM_SHARED((NPAD,), _f32),
      pltpu.VMEM_SHARED((NPAD, HID), _f32),
      pltpu.SemaphoreType.DMA,
      pltpu.SemaphoreType.DMA,
      pltpu.SemaphoreType.DMA,
      pltpu.SemaphoreType.DMA,
  ]

  @functools.partial(pl.kernel, out_type=out_type, mesh=mesh,
                     scratch_types=scratch, compiler_params=_SC_PARAMS)
  def sc_b(h1, rowf, rowi, coli, z1, z2, ones, aggp, tp, dis_out,
           ridx, cidx, didx, rowsA, rowsB, tbufA, tbufB, hloc, dloc,
           ones_v, acc, tacc, dacc, dis_sh, tstage,
           gsemA, gsemB, ssemA, ssemB):
    ci = lax.axis_index("c")
    si = lax.axis_index("s")
    wid = si * NC + ci
    t0 = si * ROWS_T
    sl = pl.ds(t0, ROWS_T)
    pltpu.sync_copy(z2.at[sl], acc.at[sl])
    pltpu.sync_copy(z1.at[sl], tacc.at[sl])
    pltpu.sync_copy(z1.at[sl], dacc.at[sl])
    pltpu.sync_copy(ones, ones_v)
    pltpu.sync_copy(rowf.at[wid], ridx)
    pltpu.sync_copy(coli.at[wid], cidx)
    plsc.subcore_barrier()

    for w_off in range(2):
      pltpu.sync_copy(rowi.at[2 * si + w_off], didx)
      for g in range(3):
        dps = [
            pltpu.async_copy(ones_v.at[j], dacc.at[didx.at[g * 27 + j]],
                             gsemA, add=True)
            for j in range(27)
        ]
        for d in dps:
          d.wait()
    plsc.subcore_barrier()

    pltpu.sync_copy(dacc.at[sl], dloc)

    def dis_step(m, carry):
      d = dloc[pl.ds(m * 16, 16)]
      rid = lax.broadcasted_iota(jnp.int32, (16,), 0) + (t0 + m * 16)
      dloc[pl.ds(m * 16, 16)] = jnp.where(rid < N, _frsqrt(d), 0.0)
      return carry

    lax.fori_loop(0, ROWS_T // 16, dis_step, 0)
    pltpu.sync_copy(dloc, dis_sh.at[sl])

    pltpu.sync_copy(h1.at[sl], hloc)

    def scale_step(r, carry):
      splat = plsc.load_gather(
          dloc, [jnp.zeros((16,), jnp.int32) + r])
      hloc[r, :] = hloc[r, :] * splat
      return carry

    lax.fori_loop(0, ROWS_T, scale_step, 0)
    pltpu.sync_copy(hloc, tstage.at[sl])
    plsc.subcore_barrier()

    def issue_gathers(c, rb, tb, gs):
      isl = pl.ds(c * CH * 128, CH * 128)
      return [
          pltpu.async_copy(tstage.at[ridx.at[isl]], rb, gs),
          pltpu.async_copy(dis_sh.at[ridx.at[isl]], tb, gs),
      ]

    def issue_scatters(c, rb, tb, ss):
      base = c * CH
      out = []
      for j in range(CH):
        out.append(pltpu.async_copy(rb.at[pl.ds(j * 128, 128)],
                                    acc.at[cidx.at[base + j]], ss, add=True))
        out.append(pltpu.async_copy(tb.at[pl.ds(j * 128, 128)],
                                    tacc.at[cidx.at[base + j]], ss, add=True))
      return out

    bufs = [(rowsA, tbufA, gsemA, ssemA), (rowsB, tbufB, gsemB, ssemB)]
    pend_g = {0: issue_gathers(0, bufs[0][0], bufs[0][1], bufs[0][2])}
    pend_s = {}
    for c in range(NCH):
      rb, tb, gs, ss = bufs[c % 2]
      if c + 1 < NCH:
        nrb, ntb, ngs, _ = bufs[(c + 1) % 2]
        if c - 1 in pend_s:
          for d in pend_s.pop(c - 1):
            d.wait()
        pend_g[c + 1] = issue_gathers(c + 1, nrb, ntb, ngs)
      for d in pend_g.pop(c):
        d.wait()
      pend_s[c] = issue_scatters(c, rb, tb, ss)
    for c in sorted(pend_s):
      for d in pend_s[c]:
        d.wait()

    plsc.subcore_barrier()
    pltpu.sync_copy(acc.at[sl], aggp.at[ci, sl])
    pltpu.sync_copy(tacc.at[sl], tp.at[ci, sl])

    @pl.when(ci == 0)
    def _():
      pltpu.sync_copy(dloc, dis_out.at[sl])

  return sc_b


def _make_sc_c():
  mesh = plsc.VectorSubcoreMesh(core_axis_name="c", subcore_axis_name="s")
  out_type = jax.ShapeDtypeStruct((NC, NPAD, HID), _f32)
  scratch = [
      pltpu.VMEM((W_E,), jnp.int32),
      pltpu.VMEM((K_W, 128), jnp.int32),
      pltpu.VMEM((CH * 128, HID), _f32),
      pltpu.VMEM((CH * 128, HID), _f32),
      pltpu.VMEM((ROWS_T, HID), _f32),
      pltpu.VMEM((ROWS_T, HID), _f32),
      pltpu.VMEM((ROWS_T,), _f32),
      pltpu.VMEM_SHARED((NPAD, HID), _f32),
      pltpu.VMEM_SHARED((NPAD, HID), _f32),
      pltpu.SemaphoreType.DMA,
      pltpu.SemaphoreType.DMA,
      pltpu.SemaphoreType.DMA,
      pltpu.SemaphoreType.DMA,
  ]

  @functools.partial(pl.kernel, out_type=out_type, mesh=mesh,
                     scratch_types=scratch, compiler_params=_SC_PARAMS)
  def sc_c(p, dis, rowf, coli, z2, aggp,
           ridx, cidx, rowsA, rowsB, g0, g1, dloc,
           acc, tstage, gsemA, gsemB, ssemA, ssemB):
    ci = lax.axis_index("c")
    si = lax.axis_index("s")
    wid = si * NC + ci
    t0 = si * ROWS_T
    sl = pl.ds(t0, ROWS_T)
    pltpu.sync_copy(z2.at[sl], acc.at[sl])
    pltpu.sync_copy(rowf.at[wid], ridx)
    pltpu.sync_copy(coli.at[wid], cidx)
    pltpu.sync_copy(p.at[0, sl], g0)
    pltpu.sync_copy(p.at[1, sl], g1)
    pltpu.sync_copy(dis.at[sl], dloc)

    def g_step(r, carry):
      splat = plsc.load_gather(dloc, [jnp.zeros((16,), jnp.int32) + r])
      v = (g0[r, :] + g1[r, :]) * splat
      g0[r, :] = jnp.maximum(v, 0.0) * splat
      return carry

    lax.fori_loop(0, ROWS_T, g_step, 0)
    pltpu.sync_copy(g0, tstage.at[sl])
    plsc.subcore_barrier()

    def issue_gathers(c, rb, gs):
      isl = pl.ds(c * CH * 128, CH * 128)
      return [pltpu.async_copy(tstage.at[ridx.at[isl]], rb, gs)]

    def issue_scatters(c, rb, ss):
      base = c * CH
      return [
          pltpu.async_copy(rb.at[pl.ds(j * 128, 128)],
                           acc.at[cidx.at[base + j]], ss, add=True)
          for j in range(CH)
      ]

    bufs = [(rowsA, gsemA, ssemA), (rowsB, gsemB, ssemB)]
    pend_g = {0: issue_gathers(0, bufs[0][0], bufs[0][1])}
    pend_s = {}
    for c in range(NCH):
      rb, gs, ss = bufs[c % 2]
      if c + 1 < NCH:
        nrb, ngs, _ = bufs[(c + 1) % 2]
        if c - 1 in pend_s:
          for d in pend_s.pop(c - 1):
            d.wait()
        pend_g[c + 1] = issue_gathers(c + 1, nrb, ngs)
      for d in pend_g.pop(c):
        d.wait()
      pend_s[c] = issue_scatters(c, rb, ss)
    for c in sorted(pend_s):
      for d in pend_s[c]:
        d.wait()

    plsc.subcore_barrier()
    pltpu.sync_copy(acc.at[sl], aggp.at[ci, sl])

  return sc_c


_sc_b = _make_sc_b()
_sc_c = _make_sc_c()


def _tc_a_body(x_blk, w1t, b1, h1_out):
  h1_out[...] = jnp.dot(x_blk[...], w1t[...],
                        preferred_element_type=_f32) + b1[...]


def _tc_a(xp, w1t, b1):
  return pl.pallas_call(
      _tc_a_body,
      grid=(GRID,),
      in_specs=[
          pl.BlockSpec((RB, IN_DIM), lambda i: (i, 0)),
          pl.BlockSpec((IN_DIM, HID), lambda i: (0, 0)),
          pl.BlockSpec((1, HID), lambda i: (0, 0)),
      ],
      out_specs=pl.BlockSpec((RB, HID), lambda i: (i, 0)),
      out_shape=jax.ShapeDtypeStruct((NPAD, HID), _f32),
  )(xp, w1t, b1)


def _tc3_body(aggp, tp, dis, w2t, b2, out):
  d = dis[...]
  a = (aggp[0] + aggp[1]) * d
  s = (tp[0] + tp[1]) * d
  o = jnp.dot(a, w2t[...], preferred_element_type=_f32) + s * b2[...]
  m = jnp.max(o, axis=1, keepdims=True)
  lse = jnp.log(jnp.sum(jnp.exp(o - m), axis=1, keepdims=True)) + m
  out[...] = o - lse


def _tc3(aggp, tp, dis, w2t, b2):
  return pl.pallas_call(
      _tc3_body,
      grid=(GRID,),
      in_specs=[
          pl.BlockSpec((NC, RB, HID), lambda i: (0, i, 0)),
          pl.BlockSpec((NC, RB, 1), lambda i: (0, i, 0)),
          pl.BlockSpec((RB, 1), lambda i: (i, 0)),
          pl.BlockSpec((HID, NCLS), lambda i: (0, 0)),
          pl.BlockSpec((1, NCLS), lambda i: (0, 0)),
      ],
      out_specs=pl.BlockSpec((RB, NCLS), lambda i: (i, 0)),
      out_shape=jax.ShapeDtypeStruct((N, NCLS), _f32),
  )(aggp, tp, dis, w2t, b2)


def kernel(x, edge_index, W1, b1, W2, b2):
  ei = edge_index.astype(jnp.int32)
  loops = jnp.arange(N, dtype=jnp.int32)
  npadfill = EP - (E + N)
  padi = N + (jnp.arange(npadfill, dtype=jnp.int32) % (NPAD - N))
  row = jnp.concatenate([ei[0], loops, padi]).reshape(NW, K_W, 128)
  rowf = lax.optimization_barrier(row).reshape(NW, W_E)
  col = jnp.concatenate([ei[1], loops, padi]).reshape(NW, K_W, 128)

  z1 = jnp.zeros((NPAD,), _f32)
  z2 = jnp.zeros((NPAD, HID), _f32)
  ones = jnp.ones((K_W, 128), _f32)
  xp = jnp.pad(x, ((0, NPAD - N), (0, 0)))

  h1 = _tc_a(xp, W1.T, b1.reshape(1, HID))
  agg1, t, dis = _sc_b(h1, rowf, row, col, z1, z2, ones)
  agg2 = _sc_c(agg1, dis, rowf, col, z2)
  return _tc3(agg2, t.reshape(NC, NPAD, 1), dis.reshape(NPAD, 1),
              W2.T, b2.reshape(1, NCLS))


# speedup vs baseline: 63.7122x; 1.1316x over previous
import functools

import jax
import jax.numpy as jnp
from jax import lax
from jax.experimental import pallas as pl
from jax.experimental.pallas import tpu as pltpu
from jax.experimental.pallas import tpu_sc as plsc

N = 10000
NPAD = 10240
E = 320000
IN_DIM = 128
HID = 16
NCLS = 40

NC = 2
NS = 16
NW = NC * NS
W_E = 10368
EP = W_E * NW
K_W = W_E // 128
CH = 9
NCH = K_W // CH
ROWS_T = NPAD // NS

RB = 1024
GRID = NPAD // RB

_f32 = jnp.float32

_SC_PARAMS = pltpu.CompilerParams(needs_layout_passes=False,
                                  use_tc_tiling_on_sc=False)


def _frsqrt(d):
  xi = plsc.bitcast(d, jnp.int32)
  y = plsc.bitcast(jnp.int32(0x5F3759DF) - lax.shift_right_logical(xi, 1),
                   _f32)
  for _ in range(3):
    y = y * (1.5 - 0.5 * d * y * y)
  return y


def _make_sc_b():
  mesh = plsc.VectorSubcoreMesh(core_axis_name="c", subcore_axis_name="s")
  out_type = (
      jax.ShapeDtypeStruct((NC, NPAD, HID), _f32),
      jax.ShapeDtypeStruct((NC, NPAD), _f32),
      jax.ShapeDtypeStruct((NPAD,), _f32),
  )
  scratch = [
      pltpu.VMEM((K_W, 128), jnp.int32),
      pltpu.VMEM((K_W, 128), jnp.int32),
      pltpu.VMEM((K_W, 128), jnp.int32),
      pltpu.VMEM((CH * 128, HID), _f32),
      pltpu.VMEM((CH * 128, HID), _f32),
      pltpu.VMEM((CH * 128,), _f32),
      pltpu.VMEM((CH * 128,), _f32),
      pltpu.VMEM((ROWS_T, HID), _f32),
      pltpu.VMEM((ROWS_T,), _f32),
      pltpu.VMEM((K_W, 128), _f32),
      pltpu.VMEM_SHARED((NPAD, HID), _f32),
      pltpu.VMEM_SHARED((NPAD,), _f32),
      pltpu.VMEM_SHARED((NPAD,), _f32),
      pltpu.VMEM_SHARED((NPAD,), _f32),
      pltpu.VMEM_SHARED((NPAD, HID), _f32),
      pltpu.SemaphoreType.DMA,
      pltpu.SemaphoreType.DMA,
      pltpu.SemaphoreType.DMA,
      pltpu.SemaphoreType.DMA,
  ]

  @functools.partial(pl.kernel, out_type=out_type, mesh=mesh,
                     scratch_types=scratch, compiler_params=_SC_PARAMS)
  def sc_b(h1, rowi, coli, z1, z2, ones, aggp, tp, dis_out,
           ridx, cidx, didx, rowsA, rowsB, tbufA, tbufB, hloc, dloc,
           ones_v, acc, tacc, dacc, dis_sh, tstage,
           gsemA, gsemB, ssemA, ssemB):
    ci = lax.axis_index("c")
    si = lax.axis_index("s")
    wid = si * NC + ci
    t0 = si * ROWS_T
    sl = pl.ds(t0, ROWS_T)
    pltpu.sync_copy(z2.at[sl], acc.at[sl])
    pltpu.sync_copy(z1.at[sl], tacc.at[sl])
    pltpu.sync_copy(z1.at[sl], dacc.at[sl])
    pltpu.sync_copy(ones, ones_v)
    pltpu.sync_copy(rowi.at[wid], ridx)
    pltpu.sync_copy(coli.at[wid], cidx)
    plsc.subcore_barrier()

    for w_off in range(2):
      pltpu.sync_copy(rowi.at[2 * si + w_off], didx)
      for g in range(3):
        dps = [
            pltpu.async_copy(ones_v.at[j], dacc.at[didx.at[g * 27 + j]],
                             gsemA, add=True)
            for j in range(27)
        ]
        for d in dps:
          d.wait()
    plsc.subcore_barrier()

    pltpu.sync_copy(dacc.at[sl], dloc)

    def dis_step(m, carry):
      d = dloc[pl.ds(m * 16, 16)]
      rid = lax.broadcasted_iota(jnp.int32, (16,), 0) + (t0 + m * 16)
      dloc[pl.ds(m * 16, 16)] = jnp.where(rid < N, _frsqrt(d), 0.0)
      return carry

    lax.fori_loop(0, ROWS_T // 16, dis_step, 0)
    pltpu.sync_copy(dloc, dis_sh.at[sl])

    pltpu.sync_copy(h1.at[sl], hloc)

    def scale_step(r, carry):
      splat = plsc.load_gather(
          dloc, [jnp.zeros((16,), jnp.int32) + r])
      hloc[r, :] = hloc[r, :] * splat
      return carry

    lax.fori_loop(0, ROWS_T, scale_step, 0)
    pltpu.sync_copy(hloc, tstage.at[sl])
    plsc.subcore_barrier()

    def issue_gathers(c, rb, tb, gs):
      base = c * CH
      out = []
      for j in range(CH):
        out.append(pltpu.async_copy(tstage.at[ridx.at[base + j]],
                                    rb.at[pl.ds(j * 128, 128)], gs))
        out.append(pltpu.async_copy(dis_sh.at[ridx.at[base + j]],
                                    tb.at[pl.ds(j * 128, 128)], gs))
      return out

    def issue_scatters(c, rb, tb, ss):
      base = c * CH
      out = []
      for j in range(CH):
        out.append(pltpu.async_copy(rb.at[pl.ds(j * 128, 128)],
                                    acc.at[cidx.at[base + j]], ss, add=True))
        out.append(pltpu.async_copy(tb.at[pl.ds(j * 128, 128)],
                                    tacc.at[cidx.at[base + j]], ss, add=True))
      return out

    bufs = [(rowsA, tbufA, gsemA, ssemA), (rowsB, tbufB, gsemB, ssemB)]
    pend_g = {0: issue_gathers(0, bufs[0][0], bufs[0][1], bufs[0][2])}
    pend_s = {}
    for c in range(NCH):
      rb, tb, gs, ss = bufs[c % 2]
      if c + 1 < NCH:
        nrb, ntb, ngs, _ = bufs[(c + 1) % 2]
        if c - 1 in pend_s:
          for d in pend_s.pop(c - 1):
            d.wait()
        pend_g[c + 1] = issue_gathers(c + 1, nrb, ntb, ngs)
      for d in pend_g.pop(c):
        d.wait()
      pend_s[c] = issue_scatters(c, rb, tb, ss)
    for c in sorted(pend_s):
      for d in pend_s[c]:
        d.wait()

    plsc.subcore_barrier()
    pltpu.sync_copy(acc.at[sl], aggp.at[ci, sl])
    pltpu.sync_copy(tacc.at[sl], tp.at[ci, sl])

    @pl.when(ci == 0)
    def _():
      pltpu.sync_copy(dloc, dis_out.at[sl])

  return sc_b


def _make_sc_c():
  mesh = plsc.VectorSubcoreMesh(core_axis_name="c", subcore_axis_name="s")
  out_type = jax.ShapeDtypeStruct((NC, NPAD, HID), _f32)
  scratch = [
      pltpu.VMEM((K_W, 128), jnp.int32),
      pltpu.VMEM((K_W, 128), jnp.int32),
      pltpu.VMEM((CH * 128, HID), _f32),
      pltpu.VMEM((CH * 128, HID), _f32),
      pltpu.VMEM((ROWS_T, HID), _f32),
      pltpu.VMEM((ROWS_T, HID), _f32),
      pltpu.VMEM((ROWS_T,), _f32),
      pltpu.VMEM_SHARED((NPAD, HID), _f32),
      pltpu.VMEM_SHARED((NPAD, HID), _f32),
      pltpu.SemaphoreType.DMA,
      pltpu.SemaphoreType.DMA,
      pltpu.SemaphoreType.DMA,
      pltpu.SemaphoreType.DMA,
  ]

  @functools.partial(pl.kernel, out_type=out_type, mesh=mesh,
                     scratch_types=scratch, compiler_params=_SC_PARAMS)
  def sc_c(p, dis, rowi, coli, z2, aggp,
           ridx, cidx, rowsA, rowsB, g0, g1, dloc,
           acc, tstage, gsemA, gsemB, ssemA, ssemB):
    ci = lax.axis_index("c")
    si = lax.axis_index("s")
    wid = si * NC + ci
    t0 = si * ROWS_T
    sl = pl.ds(t0, ROWS_T)
    pltpu.sync_copy(z2.at[sl], acc.at[sl])
    pltpu.sync_copy(rowi.at[wid], ridx)
    pltpu.sync_copy(coli.at[wid], cidx)
    pltpu.sync_copy(p.at[0, sl], g0)
    pltpu.sync_copy(p.at[1, sl], g1)
    pltpu.sync_copy(dis.at[sl], dloc)

    def g_step(r, carry):
      splat = plsc.load_gather(dloc, [jnp.zeros((16,), jnp.int32) + r])
      v = (g0[r, :] + g1[r, :]) * splat
      g0[r, :] = jnp.maximum(v, 0.0) * splat
      return carry

    lax.fori_loop(0, ROWS_T, g_step, 0)
    pltpu.sync_copy(g0, tstage.at[sl])
    plsc.subcore_barrier()

    def issue_gathers(c, rb, gs):
      base = c * CH
      return [
          pltpu.async_copy(tstage.at[ridx.at[base + j]],
                           rb.at[pl.ds(j * 128, 128)], gs)
          for j in range(CH)
      ]

    def issue_scatters(c, rb, ss):
      base = c * CH
      return [
          pltpu.async_copy(rb.at[pl.ds(j * 128, 128)],
                           acc.at[cidx.at[base + j]], ss, add=True)
          for j in range(CH)
      ]

    bufs = [(rowsA, gsemA, ssemA), (rowsB, gsemB, ssemB)]
    pend_g = {0: issue_gathers(0, bufs[0][0], bufs[0][1])}
    pend_s = {}
    for c in range(NCH):
      rb, gs, ss = bufs[c % 2]
      if c + 1 < NCH:
        nrb, ngs, _ = bufs[(c + 1) % 2]
        if c - 1 in pend_s:
          for d in pend_s.pop(c - 1):
            d.wait()
        pend_g[c + 1] = issue_gathers(c + 1, nrb, ngs)
      for d in pend_g.pop(c):
        d.wait()
      pend_s[c] = issue_scatters(c, rb, ss)
    for c in sorted(pend_s):
      for d in pend_s[c]:
        d.wait()

    plsc.subcore_barrier()
    pltpu.sync_copy(acc.at[sl], aggp.at[ci, sl])

  return sc_c


_sc_b = _make_sc_b()
_sc_c = _make_sc_c()


def _tc_a_body(x_blk, w1t, b1, h1_out):
  i = pl.program_id(0)
  h = jnp.dot(x_blk[...], w1t[...], preferred_element_type=_f32) + b1[...]
  rid = lax.broadcasted_iota(jnp.int32, (RB, 1), 0) + i * RB
  h1_out[...] = jnp.where(rid < N, h, 0.0)


def _tc_a(x, w1t, b1):
  return pl.pallas_call(
      _tc_a_body,
      grid=(GRID,),
      in_specs=[
          pl.BlockSpec((RB, IN_DIM), lambda i: (i, 0)),
          pl.BlockSpec((IN_DIM, HID), lambda i: (0, 0)),
          pl.BlockSpec((1, HID), lambda i: (0, 0)),
      ],
      out_specs=pl.BlockSpec((RB, HID), lambda i: (i, 0)),
      out_shape=jax.ShapeDtypeStruct((NPAD, HID), _f32),
  )(x, w1t, b1)


def _tc3_body(aggp, tp, dis, w2t, b2, out):
  d = dis[...]
  a = (aggp[0] + aggp[1]) * d
  s = (tp[0] + tp[1]) * d
  o = jnp.dot(a, w2t[...], preferred_element_type=_f32) + s * b2[...]
  m = jnp.max(o, axis=1, keepdims=True)
  lse = jnp.log(jnp.sum(jnp.exp(o - m), axis=1, keepdims=True)) + m
  out[...] = o - lse


def _tc3(aggp, tp, dis, w2t, b2):
  return pl.pallas_call(
      _tc3_body,
      grid=(GRID,),
      in_specs=[
          pl.BlockSpec((NC, RB, HID), lambda i: (0, i, 0)),
          pl.BlockSpec((NC, RB, 1), lambda i: (0, i, 0)),
          pl.BlockSpec((RB, 1), lambda i: (i, 0)),
          pl.BlockSpec((HID, NCLS), lambda i: (0, 0)),
          pl.BlockSpec((1, NCLS), lambda i: (0, 0)),
      ],
      out_specs=pl.BlockSpec((RB, NCLS), lambda i: (i, 0)),
      out_shape=jax.ShapeDtypeStruct((N, NCLS), _f32),
  )(aggp, tp, dis, w2t, b2)


def kernel(x, edge_index, W1, b1, W2, b2):
  ei = edge_index.astype(jnp.int32)
  loops = jnp.arange(N, dtype=jnp.int32)
  npadfill = EP - (E + N)
  padi = N + (jnp.arange(npadfill, dtype=jnp.int32) % (NPAD - N))
  loops2 = jnp.stack([loops, loops])
  padi2 = jnp.stack([padi, padi])
  ei_all = jnp.concatenate([ei, loops2, padi2], axis=1)
  row = ei_all[0].reshape(NW, K_W, 128)
  col = ei_all[1].reshape(NW, K_W, 128)

  z1 = jnp.zeros((NPAD,), _f32)
  z2 = jnp.zeros((NPAD, HID), _f32)
  ones = jnp.ones((K_W, 128), _f32)

  h1 = _tc_a(x, W1.T, b1.reshape(1, HID))
  agg1, t, dis = _sc_b(h1, row, col, z1, z2, ones)
  agg2 = _sc_c(agg1, dis, row, col, z2)
  return _tc3(agg2, t.reshape(NC, NPAD, 1), dis.reshape(NPAD, 1),
              W2.T, b2.reshape(1, NCLS))


# speedup vs baseline: 68.5412x; 1.0758x over previous
import functools

import jax
import jax.numpy as jnp
from jax import lax
from jax.experimental import pallas as pl
from jax.experimental.pallas import tpu as pltpu
from jax.experimental.pallas import tpu_sc as plsc

N = 10000
NPAD = 10240
E = 320000
IN_DIM = 128
HID = 16
NCLS = 40

NC = 2
NS = 16
NW = NC * NS
W_E = 10368
EP = W_E * NW
K_W = W_E // 128
CH = 9
NCH = K_W // CH
ROWS_T = NPAD // NS

RB = 1024
GRID = NPAD // RB

_f32 = jnp.float32

_SC_PARAMS = pltpu.CompilerParams(needs_layout_passes=False,
                                  use_tc_tiling_on_sc=False)


def _frsqrt(d):
  xi = plsc.bitcast(d, jnp.int32)
  y = plsc.bitcast(jnp.int32(0x5F3759DF) - lax.shift_right_logical(xi, 1),
                   _f32)
  for _ in range(3):
    y = y * (1.5 - 0.5 * d * y * y)
  return y


def _make_sc_deg():
  mesh = plsc.VectorSubcoreMesh(core_axis_name="c", subcore_axis_name="s")
  scratch = [
      pltpu.VMEM((K_W, 128), jnp.int32),
      pltpu.VMEM((K_W, 128), _f32),
      pltpu.VMEM_SHARED((NPAD,), _f32),
      pltpu.SemaphoreType.DMA,
  ]

  @functools.partial(
      pl.kernel,
      out_type=jax.ShapeDtypeStruct((NC, NPAD), _f32),
      mesh=mesh,
      scratch_types=scratch,
      compiler_params=_SC_PARAMS,
  )
  def sc_deg(rowi, z1, ones, degp, didx, ones_v, dacc, dsem):
    ci = lax.axis_index("c")
    si = lax.axis_index("s")
    wid = si * NC + ci
    t0 = si * ROWS_T
    sl = pl.ds(t0, ROWS_T)
    pltpu.sync_copy(z1.at[sl], dacc.at[sl])
    pltpu.sync_copy(ones, ones_v)
    pltpu.sync_copy(rowi.at[wid], didx)
    plsc.subcore_barrier()
    for g in range(3):
      dps = [
          pltpu.async_copy(ones_v.at[j], dacc.at[didx.at[g * 27 + j]],
                           dsem, add=True)
          for j in range(27)
      ]
      for d in dps:
        d.wait()
    plsc.subcore_barrier()
    pltpu.sync_copy(dacc.at[sl], degp.at[ci, sl])

  return sc_deg


def _make_sc_b():
  mesh = plsc.VectorSubcoreMesh(core_axis_name="c", subcore_axis_name="s")
  out_type = (
      jax.ShapeDtypeStruct((NC, NPAD, HID), _f32),
      jax.ShapeDtypeStruct((NC, NPAD), _f32),
      jax.ShapeDtypeStruct((NPAD,), _f32),
  )
  scratch = [
      pltpu.VMEM((K_W, 128), jnp.int32),
      pltpu.VMEM((K_W, 128), jnp.int32),
      pltpu.VMEM((CH * 128, HID), _f32),
      pltpu.VMEM((CH * 128, HID), _f32),
      pltpu.VMEM((CH * 128,), _f32),
      pltpu.VMEM((CH * 128,), _f32),
      pltpu.VMEM((ROWS_T, HID), _f32),
      pltpu.VMEM((ROWS_T,), _f32),
      pltpu.VMEM((ROWS_T,), _f32),
      pltpu.VMEM_SHARED((NPAD, HID), _f32),
      pltpu.VMEM_SHARED((NPAD,), _f32),
      pltpu.VMEM_SHARED((NPAD,), _f32),
      pltpu.VMEM_SHARED((NPAD, HID), _f32),
      pltpu.SemaphoreType.DMA,
      pltpu.SemaphoreType.DMA,
      pltpu.SemaphoreType.DMA,
      pltpu.SemaphoreType.DMA,
  ]

  @functools.partial(pl.kernel, out_type=out_type, mesh=mesh,
                     scratch_types=scratch, compiler_params=_SC_PARAMS)
  def sc_b(h1, degp, rowi, coli, z1, z2, aggp, tp, dis_out,
           ridx, cidx, rowsA, rowsB, tbufA, tbufB, hloc, dloc, dloc2,
           acc, tacc, dis_sh, tstage,
           gsemA, gsemB, ssemA, ssemB):
    ci = lax.axis_index("c")
    si = lax.axis_index("s")
    wid = si * NC + ci
    t0 = si * ROWS_T
    sl = pl.ds(t0, ROWS_T)
    pltpu.sync_copy(z2.at[sl], acc.at[sl])
    pltpu.sync_copy(z1.at[sl], tacc.at[sl])
    pltpu.sync_copy(rowi.at[wid], ridx)
    pltpu.sync_copy(coli.at[wid], cidx)
    pltpu.sync_copy(h1.at[sl], hloc)

    pltpu.sync_copy(degp.at[0, sl], dloc)
    pltpu.sync_copy(degp.at[1, sl], dloc2)

    def dis_step(m, carry):
      d = dloc[pl.ds(m * 16, 16)] + dloc2[pl.ds(m * 16, 16)]
      rid = lax.broadcasted_iota(jnp.int32, (16,), 0) + (t0 + m * 16)
      dloc[pl.ds(m * 16, 16)] = jnp.where(rid < N, _frsqrt(d), 0.0)
      return carry

    lax.fori_loop(0, ROWS_T // 16, dis_step, 0)
    pltpu.sync_copy(dloc, dis_sh.at[sl])

    def scale_step(r, carry):
      splat = plsc.load_gather(
          dloc, [jnp.zeros((16,), jnp.int32) + r])
      hloc[r, :] = hloc[r, :] * splat
      return carry

    lax.fori_loop(0, ROWS_T, scale_step, 0)
    pltpu.sync_copy(hloc, tstage.at[sl])
    plsc.subcore_barrier()

    def issue_gathers(c, rb, tb, gs):
      base = c * CH
      out = []
      for j in range(CH):
        out.append(pltpu.async_copy(tstage.at[ridx.at[base + j]],
                                    rb.at[pl.ds(j * 128, 128)], gs))
        out.append(pltpu.async_copy(dis_sh.at[ridx.at[base + j]],
                                    tb.at[pl.ds(j * 128, 128)], gs))
      return out

    def issue_scatters(c, rb, tb, ss):
      base = c * CH
      out = []
      for j in range(CH):
        out.append(pltpu.async_copy(rb.at[pl.ds(j * 128, 128)],
                                    acc.at[cidx.at[base + j]], ss, add=True))
        out.append(pltpu.async_copy(tb.at[pl.ds(j * 128, 128)],
                                    tacc.at[cidx.at[base + j]], ss, add=True))
      return out

    bufs = [(rowsA, tbufA, gsemA, ssemA), (rowsB, tbufB, gsemB, ssemB)]
    pend_g = {0: issue_gathers(0, bufs[0][0], bufs[0][1], bufs[0][2])}
    pend_s = {}
    for c in range(NCH):
      rb, tb, gs, ss = bufs[c % 2]
      if c + 1 < NCH:
        nrb, ntb, ngs, _ = bufs[(c + 1) % 2]
        if c - 1 in pend_s:
          for d in pend_s.pop(c - 1):
            d.wait()
        pend_g[c + 1] = issue_gathers(c + 1, nrb, ntb, ngs)
      for d in pend_g.pop(c):
        d.wait()
      pend_s[c] = issue_scatters(c, rb, tb, ss)
    for c in sorted(pend_s):
      for d in pend_s[c]:
        d.wait()

    plsc.subcore_barrier()
    pltpu.sync_copy(acc.at[sl], aggp.at[ci, sl])
    pltpu.sync_copy(tacc.at[sl], tp.at[ci, sl])

    @pl.when(ci == 0)
    def _():
      pltpu.sync_copy(dloc, dis_out.at[sl])

  return sc_b


def _make_sc_c():
  mesh = plsc.VectorSubcoreMesh(core_axis_name="c", subcore_axis_name="s")
  out_type = jax.ShapeDtypeStruct((NC, NPAD, HID), _f32)
  scratch = [
      pltpu.VMEM((K_W, 128), jnp.int32),
      pltpu.VMEM((K_W, 128), jnp.int32),
      pltpu.VMEM((CH * 128, HID), _f32),
      pltpu.VMEM((CH * 128, HID), _f32),
      pltpu.VMEM((ROWS_T, HID), _f32),
      pltpu.VMEM((ROWS_T, HID), _f32),
      pltpu.VMEM((ROWS_T,), _f32),
      pltpu.VMEM_SHARED((NPAD, HID), _f32),
      pltpu.VMEM_SHARED((NPAD, HID), _f32),
      pltpu.SemaphoreType.DMA,
      pltpu.SemaphoreType.DMA,
      pltpu.SemaphoreType.DMA,
      pltpu.SemaphoreType.DMA,
  ]

  @functools.partial(pl.kernel, out_type=out_type, mesh=mesh,
                     scratch_types=scratch, compiler_params=_SC_PARAMS)
  def sc_c(p, dis, rowi, coli, z2, aggp,
           ridx, cidx, rowsA, rowsB, g0, g1, dloc,
           acc, tstage, gsemA, gsemB, ssemA, ssemB):
    ci = lax.axis_index("c")
    si = lax.axis_index("s")
    wid = si * NC + ci
    t0 = si * ROWS_T
    sl = pl.ds(t0, ROWS_T)
    pltpu.sync_copy(z2.at[sl], acc.at[sl])
    pltpu.sync_copy(rowi.at[wid], ridx)
    pltpu.sync_copy(coli.at[wid], cidx)
    pltpu.sync_copy(p.at[0, sl], g0)
    pltpu.sync_copy(p.at[1, sl], g1)
    pltpu.sync_copy(dis.at[sl], dloc)

    def g_step(r, carry):
      splat = plsc.load_gather(dloc, [jnp.zeros((16,), jnp.int32) + r])
      v = (g0[r, :] + g1[r, :]) * splat
      g0[r, :] = jnp.maximum(v, 0.0) * splat
      return carry

    lax.fori_loop(0, ROWS_T, g_step, 0)
    pltpu.sync_copy(g0, tstage.at[sl])
    plsc.subcore_barrier()

    def issue_gathers(c, rb, gs):
      base = c * CH
      return [
          pltpu.async_copy(tstage.at[ridx.at[base + j]],
                           rb.at[pl.ds(j * 128, 128)], gs)
          for j in range(CH)
      ]

    def issue_scatters(c, rb, ss):
      base = c * CH
      return [
          pltpu.async_copy(rb.at[pl.ds(j * 128, 128)],
                           acc.at[cidx.at[base + j]], ss, add=True)
          for j in range(CH)
      ]

    bufs = [(rowsA, gsemA, ssemA), (rowsB, gsemB, ssemB)]
    pend_g = {0: issue_gathers(0, bufs[0][0], bufs[0][1])}
    pend_s = {}
    for c in range(NCH):
      rb, gs, ss = bufs[c % 2]
      if c + 1 < NCH:
        nrb, ngs, _ = bufs[(c + 1) % 2]
        if c - 1 in pend_s:
          for d in pend_s.pop(c - 1):
            d.wait()
        pend_g[c + 1] = issue_gathers(c + 1, nrb, ngs)
      for d in pend_g.pop(c):
        d.wait()
      pend_s[c] = issue_scatters(c, rb, ss)
    for c in sorted(pend_s):
      for d in pend_s[c]:
        d.wait()

    plsc.subcore_barrier()
    pltpu.sync_copy(acc.at[sl], aggp.at[ci, sl])

  return sc_c


_sc_deg = _make_sc_deg()
_sc_b = _make_sc_b()
_sc_c = _make_sc_c()


def _tc_a_body(x_blk, w1t, b1, h1_out):
  i = pl.program_id(0)
  h = jnp.dot(x_blk[...], w1t[...], preferred_element_type=_f32) + b1[...]
  rid = lax.broadcasted_iota(jnp.int32, (RB, 1), 0) + i * RB
  h1_out[...] = jnp.where(rid < N, h, 0.0)


def _tc_a(x, w1t, b1):
  return pl.pallas_call(
      _tc_a_body,
      grid=(GRID,),
      in_specs=[
          pl.BlockSpec((RB, IN_DIM), lambda i: (i, 0)),
          pl.BlockSpec((IN_DIM, HID), lambda i: (0, 0)),
          pl.BlockSpec((1, HID), lambda i: (0, 0)),
      ],
      out_specs=pl.BlockSpec((RB, HID), lambda i: (i, 0)),
      out_shape=jax.ShapeDtypeStruct((NPAD, HID), _f32),
  )(x, w1t, b1)


def _tc3_body(aggp, tp, dis, w2t, b2, out):
  d = dis[...]
  a = (aggp[0] + aggp[1]) * d
  s = (tp[0] + tp[1]) * d
  o = jnp.dot(a, w2t[...], preferred_element_type=_f32) + s * b2[...]
  m = jnp.max(o, axis=1, keepdims=True)
  lse = jnp.log(jnp.sum(jnp.exp(o - m), axis=1, keepdims=True)) + m
  out[...] = o - lse


def _tc3(aggp, tp, dis, w2t, b2):
  return pl.pallas_call(
      _tc3_body,
      grid=(GRID,),
      in_specs=[
          pl.BlockSpec((NC, RB, HID), lambda i: (0, i, 0)),
          pl.BlockSpec((NC, RB, 1), lambda i: (0, i, 0)),
          pl.BlockSpec((RB, 1), lambda i: (i, 0)),
          pl.BlockSpec((HID, NCLS), lambda i: (0, 0)),
          pl.BlockSpec((1, NCLS), lambda i: (0, 0)),
      ],
      out_specs=pl.BlockSpec((RB, NCLS), lambda i: (i, 0)),
      out_shape=jax.ShapeDtypeStruct((N, NCLS), _f32),
  )(aggp, tp, dis, w2t, b2)


def kernel(x, edge_index, W1, b1, W2, b2):
  ei = edge_index.astype(jnp.int32)
  loops = jnp.arange(N, dtype=jnp.int32)
  npadfill = EP - (E + N)
  padi = N + (jnp.arange(npadfill, dtype=jnp.int32) % (NPAD - N))
  loops2 = jnp.stack([loops, loops])
  padi2 = jnp.stack([padi, padi])
  ei_all = jnp.concatenate([ei, loops2, padi2], axis=1)
  row = ei_all[0].reshape(NW, K_W, 128)
  col = ei_all[1].reshape(NW, K_W, 128)

  z1 = jnp.zeros((NPAD,), _f32)
  z2 = jnp.zeros((NPAD, HID), _f32)
  ones = jnp.ones((K_W, 128), _f32)

  degp = _sc_deg(row, z1, ones)
  h1 = _tc_a(x, W1.T, b1.reshape(1, HID))
  agg1, t, dis = _sc_b(h1, degp, row, col, z1, z2)
  agg2 = _sc_c(agg1, dis, row, col, z2)
  return _tc3(agg2, t.reshape(NC, NPAD, 1), dis.reshape(NPAD, 1),
              W2.T, b2.reshape(1, NCLS))


# speedup vs baseline: 73.6343x; 1.0743x over previous
import functools

import jax
import jax.numpy as jnp
from jax import lax
from jax.experimental import pallas as pl
from jax.experimental.pallas import tpu as pltpu
from jax.experimental.pallas import tpu_sc as plsc

N = 10000
NPAD = 10240
E = 320000
IN_DIM = 128
HID = 16
NCLS = 40

NC = 2
NS = 16
NW = NC * NS
W_E = 10368
EP = W_E * NW
K_W = W_E // 128
CH = 9
NCH = K_W // CH
ROWS_T = NPAD // NS

RB = 1024
GRID = NPAD // RB

_f32 = jnp.float32

_SC_PARAMS = pltpu.CompilerParams(needs_layout_passes=False,
                                  use_tc_tiling_on_sc=False)


def _frsqrt(d):
  xi = plsc.bitcast(d, jnp.int32)
  y = plsc.bitcast(jnp.int32(0x5F3759DF) - lax.shift_right_logical(xi, 1),
                   _f32)
  for _ in range(3):
    y = y * (1.5 - 0.5 * d * y * y)
  return y


def _make_sc_deg():
  mesh = plsc.VectorSubcoreMesh(core_axis_name="c", subcore_axis_name="s")
  scratch = [
      pltpu.VMEM((K_W, 128), jnp.int32),
      pltpu.VMEM((K_W, 128), _f32),
      pltpu.VMEM_SHARED((NPAD,), _f32),
      pltpu.SemaphoreType.DMA,
  ]

  @functools.partial(
      pl.kernel,
      out_type=jax.ShapeDtypeStruct((NC, NPAD), _f32),
      mesh=mesh,
      scratch_types=scratch,
      compiler_params=_SC_PARAMS,
  )
  def sc_deg(rowi, z1, ones, degp, didx, ones_v, dacc, dsem):
    ci = lax.axis_index("c")
    si = lax.axis_index("s")
    wid = si * NC + ci
    t0 = si * ROWS_T
    sl = pl.ds(t0, ROWS_T)
    pltpu.sync_copy(z1.at[sl], dacc.at[sl])
    pltpu.sync_copy(ones, ones_v)
    pltpu.sync_copy(rowi.at[wid], didx)
    plsc.subcore_barrier()
    for g in range(3):
      dps = [
          pltpu.async_copy(ones_v.at[j], dacc.at[didx.at[g * 27 + j]],
                           dsem, add=True)
          for j in range(27)
      ]
      for d in dps:
        d.wait()
    plsc.subcore_barrier()
    pltpu.sync_copy(dacc.at[sl], degp.at[ci, sl])

  return sc_deg


def _make_sc_b():
  mesh = plsc.VectorSubcoreMesh(core_axis_name="c", subcore_axis_name="s")
  out_type = (
      jax.ShapeDtypeStruct((NC, NPAD, HID), _f32),
      jax.ShapeDtypeStruct((NC, NPAD), _f32),
      jax.ShapeDtypeStruct((NPAD,), _f32),
  )
  scratch = [
      pltpu.VMEM((K_W, 128), jnp.int32),
      pltpu.VMEM((K_W, 128), jnp.int32),
      pltpu.VMEM((CH * 128, HID), _f32),
      pltpu.VMEM((CH * 128, HID), _f32),
      pltpu.VMEM((CH * 128,), _f32),
      pltpu.VMEM((CH * 128,), _f32),
      pltpu.VMEM((ROWS_T, HID), _f32),
      pltpu.VMEM((ROWS_T,), _f32),
      pltpu.VMEM((ROWS_T,), _f32),
      pltpu.VMEM_SHARED((NPAD, HID), _f32),
      pltpu.VMEM_SHARED((NPAD,), _f32),
      pltpu.VMEM_SHARED((NPAD,), _f32),
      pltpu.VMEM_SHARED((NPAD, HID), _f32),
      pltpu.SemaphoreType.DMA,
      pltpu.SemaphoreType.DMA,
      pltpu.SemaphoreType.DMA,
      pltpu.SemaphoreType.DMA,
  ]

  @functools.partial(pl.kernel, out_type=out_type, mesh=mesh,
                     scratch_types=scratch, compiler_params=_SC_PARAMS)
  def sc_b(h1, degp, rowi, coli, z1, z2, aggp, tp, dis_out,
           ridx, cidx, rowsA, rowsB, tbufA, tbufB, hloc, dloc, dloc2,
           acc, tacc, dis_sh, tstage,
           gsemA, gsemB, ssemA, ssemB):
    ci = lax.axis_index("c")
    si = lax.axis_index("s")
    wid = si * NC + ci
    t0 = si * ROWS_T
    sl = pl.ds(t0, ROWS_T)
    pltpu.sync_copy(z2.at[sl], acc.at[sl])
    pltpu.sync_copy(z1.at[sl], tacc.at[sl])
    pltpu.sync_copy(rowi.at[wid], ridx)
    pltpu.sync_copy(coli.at[wid], cidx)
    pltpu.sync_copy(h1.at[sl], hloc)

    pltpu.sync_copy(degp.at[0, sl], dloc)
    pltpu.sync_copy(degp.at[1, sl], dloc2)

    @functools.partial(plsc.parallel_loop, 0, ROWS_T // 16, unroll=4)
    def dis_step(m):
      d = dloc[pl.ds(m * 16, 16)] + dloc2[pl.ds(m * 16, 16)]
      rid = lax.broadcasted_iota(jnp.int32, (16,), 0) + (t0 + m * 16)
      dloc[pl.ds(m * 16, 16)] = jnp.where(rid < N, _frsqrt(d), 0.0)
    pltpu.sync_copy(dloc, dis_sh.at[sl])

    @functools.partial(plsc.parallel_loop, 0, ROWS_T, unroll=8)
    def scale_step(r):
      splat = plsc.load_gather(
          dloc, [jnp.zeros((16,), jnp.int32) + r])
      hloc[r, :] = hloc[r, :] * splat
    pltpu.sync_copy(hloc, tstage.at[sl])
    plsc.subcore_barrier()

    def issue_gathers(c, rb, tb, gs):
      base = c * CH
      out = []
      for j in range(CH):
        out.append(pltpu.async_copy(tstage.at[ridx.at[base + j]],
                                    rb.at[pl.ds(j * 128, 128)], gs))
        out.append(pltpu.async_copy(dis_sh.at[ridx.at[base + j]],
                                    tb.at[pl.ds(j * 128, 128)], gs))
      return out

    def issue_scatters(c, rb, tb, ss):
      base = c * CH
      out = []
      for j in range(CH):
        out.append(pltpu.async_copy(rb.at[pl.ds(j * 128, 128)],
                                    acc.at[cidx.at[base + j]], ss, add=True))
        out.append(pltpu.async_copy(tb.at[pl.ds(j * 128, 128)],
                                    tacc.at[cidx.at[base + j]], ss, add=True))
      return out

    bufs = [(rowsA, tbufA, gsemA, ssemA), (rowsB, tbufB, gsemB, ssemB)]
    pend_g = {0: issue_gathers(0, bufs[0][0], bufs[0][1], bufs[0][2])}
    pend_s = {}
    for c in range(NCH):
      rb, tb, gs, ss = bufs[c % 2]
      if c + 1 < NCH:
        nrb, ntb, ngs, _ = bufs[(c + 1) % 2]
        if c - 1 in pend_s:
          for d in pend_s.pop(c - 1):
            d.wait()
        pend_g[c + 1] = issue_gathers(c + 1, nrb, ntb, ngs)
      for d in pend_g.pop(c):
        d.wait()
      pend_s[c] = issue_scatters(c, rb, tb, ss)
    for c in sorted(pend_s):
      for d in pend_s[c]:
        d.wait()

    plsc.subcore_barrier()
    pltpu.sync_copy(acc.at[sl], aggp.at[ci, sl])
    pltpu.sync_copy(tacc.at[sl], tp.at[ci, sl])

    @pl.when(ci == 0)
    def _():
      pltpu.sync_copy(dloc, dis_out.at[sl])

  return sc_b


def _make_sc_c():
  mesh = plsc.VectorSubcoreMesh(core_axis_name="c", subcore_axis_name="s")
  out_type = jax.ShapeDtypeStruct((NC, NPAD, HID), _f32)
  scratch = [
      pltpu.VMEM((K_W, 128), jnp.int32),
      pltpu.VMEM((K_W, 128), jnp.int32),
      pltpu.VMEM((CH * 128, HID), _f32),
      pltpu.VMEM((CH * 128, HID), _f32),
      pltpu.VMEM((ROWS_T, HID), _f32),
      pltpu.VMEM((ROWS_T, HID), _f32),
      pltpu.VMEM((ROWS_T,), _f32),
      pltpu.VMEM_SHARED((NPAD, HID), _f32),
      pltpu.VMEM_SHARED((NPAD, HID), _f32),
      pltpu.SemaphoreType.DMA,
      pltpu.SemaphoreType.DMA,
      pltpu.SemaphoreType.DMA,
      pltpu.SemaphoreType.DMA,
  ]

  @functools.partial(pl.kernel, out_type=out_type, mesh=mesh,
                     scratch_types=scratch, compiler_params=_SC_PARAMS)
  def sc_c(p, dis, rowi, coli, z2, aggp,
           ridx, cidx, rowsA, rowsB, g0, g1, dloc,
           acc, tstage, gsemA, gsemB, ssemA, ssemB):
    ci = lax.axis_index("c")
    si = lax.axis_index("s")
    wid = si * NC + ci
    t0 = si * ROWS_T
    sl = pl.ds(t0, ROWS_T)
    pltpu.sync_copy(z2.at[sl], acc.at[sl])
    pltpu.sync_copy(rowi.at[wid], ridx)
    pltpu.sync_copy(coli.at[wid], cidx)
    pltpu.sync_copy(p.at[0, sl], g0)
    pltpu.sync_copy(p.at[1, sl], g1)
    pltpu.sync_copy(dis.at[sl], dloc)

    @functools.partial(plsc.parallel_loop, 0, ROWS_T, unroll=8)
    def g_step(r):
      splat = plsc.load_gather(dloc, [jnp.zeros((16,), jnp.int32) + r])
      v = (g0[r, :] + g1[r, :]) * splat
      g0[r, :] = jnp.maximum(v, 0.0) * splat
    pltpu.sync_copy(g0, tstage.at[sl])
    plsc.subcore_barrier()

    def issue_gathers(c, rb, gs):
      base = c * CH
      return [
          pltpu.async_copy(tstage.at[ridx.at[base + j]],
                           rb.at[pl.ds(j * 128, 128)], gs)
          for j in range(CH)
      ]

    def issue_scatters(c, rb, ss):
      base = c * CH
      return [
          pltpu.async_copy(rb.at[pl.ds(j * 128, 128)],
                           acc.at[cidx.at[base + j]], ss, add=True)
          for j in range(CH)
      ]

    bufs = [(rowsA, gsemA, ssemA), (rowsB, gsemB, ssemB)]
    pend_g = {0: issue_gathers(0, bufs[0][0], bufs[0][1])}
    pend_s = {}
    for c in range(NCH):
      rb, gs, ss = bufs[c % 2]
      if c + 1 < NCH:
        nrb, ngs, _ = bufs[(c + 1) % 2]
        if c - 1 in pend_s:
          for d in pend_s.pop(c - 1):
            d.wait()
        pend_g[c + 1] = issue_gathers(c + 1, nrb, ngs)
      for d in pend_g.pop(c):
        d.wait()
      pend_s[c] = issue_scatters(c, rb, ss)
    for c in sorted(pend_s):
      for d in pend_s[c]:
        d.wait()

    plsc.subcore_barrier()
    pltpu.sync_copy(acc.at[sl], aggp.at[ci, sl])

  return sc_c


_sc_deg = _make_sc_deg()
_sc_b = _make_sc_b()
_sc_c = _make_sc_c()


def _tc_a_body(x_blk, w1t, b1, h1_out):
  i = pl.program_id(0)
  h = jnp.dot(x_blk[...], w1t[...], preferred_element_type=_f32) + b1[...]
  rid = lax.broadcasted_iota(jnp.int32, (RB, 1), 0) + i * RB
  h1_out[...] = jnp.where(rid < N, h, 0.0)


def _tc_a(x, w1t, b1):
  return pl.pallas_call(
      _tc_a_body,
      grid=(GRID,),
      in_specs=[
          pl.BlockSpec((RB, IN_DIM), lambda i: (i, 0)),
          pl.BlockSpec((IN_DIM, HID), lambda i: (0, 0)),
          pl.BlockSpec((1, HID), lambda i: (0, 0)),
      ],
      out_specs=pl.BlockSpec((RB, HID), lambda i: (i, 0)),
      out_shape=jax.ShapeDtypeStruct((NPAD, HID), _f32),
  )(x, w1t, b1)


def _tc3_body(aggp, tp, dis, w2t, b2, out):
  d = dis[...]
  a = (aggp[0] + aggp[1]) * d
  s = (tp[0] + tp[1]) * d
  o = jnp.dot(a, w2t[...], preferred_element_type=_f32) + s * b2[...]
  m = jnp.max(o, axis=1, keepdims=True)
  lse = jnp.log(jnp.sum(jnp.exp(o - m), axis=1, keepdims=True)) + m
  out[...] = o - lse


def _tc3(aggp, tp, dis, w2t, b2):
  return pl.pallas_call(
      _tc3_body,
      grid=(GRID,),
      in_specs=[
          pl.BlockSpec((NC, RB, HID), lambda i: (0, i, 0)),
          pl.BlockSpec((NC, RB, 1), lambda i: (0, i, 0)),
          pl.BlockSpec((RB, 1), lambda i: (i, 0)),
          pl.BlockSpec((HID, NCLS), lambda i: (0, 0)),
          pl.BlockSpec((1, NCLS), lambda i: (0, 0)),
      ],
      out_specs=pl.BlockSpec((RB, NCLS), lambda i: (i, 0)),
      out_shape=jax.ShapeDtypeStruct((N, NCLS), _f32),
  )(aggp, tp, dis, w2t, b2)


def kernel(x, edge_index, W1, b1, W2, b2):
  ei = edge_index.astype(jnp.int32)
  loops = jnp.arange(N, dtype=jnp.int32)
  npadfill = EP - (E + N)
  padi = N + (jnp.arange(npadfill, dtype=jnp.int32) % (NPAD - N))
  loops2 = jnp.stack([loops, loops])
  padi2 = jnp.stack([padi, padi])
  ei_all = jnp.concatenate([ei, loops2, padi2], axis=1)
  row = ei_all[0].reshape(NW, K_W, 128)
  col = ei_all[1].reshape(NW, K_W, 128)

  z1 = jnp.zeros((NPAD,), _f32)
  z2 = jnp.zeros((NPAD, HID), _f32)
  ones = jnp.ones((K_W, 128), _f32)

  degp = _sc_deg(row, z1, ones)
  h1 = _tc_a(x, W1.T, b1.reshape(1, HID))
  agg1, t, dis = _sc_b(h1, degp, row, col, z1, z2)
  agg2 = _sc_c(agg1, dis, row, col, z2)
  return _tc3(agg2, t.reshape(NC, NPAD, 1), dis.reshape(NPAD, 1),
              W2.T, b2.reshape(1, NCLS))
